# baseline (speedup 1.0000x reference)
import functools

import jax
import jax.numpy as jnp
from jax import lax
from jax.experimental import pallas as pl
from jax.experimental.pallas import tpu as pltpu
from jax.experimental.pallas import tpu_sc as plsc

N_NODES = 10000
N_EDGES = 160000
N_UNARY = 128

NTILES = 16
SPAN = 640
SPAN_LAST = N_NODES - 15 * SPAN
NPAD = NTILES * SPAN
TFLAT = NPAD * 4
P1C = 16
EPT = N_EDGES // NTILES
EC = 400
NCHUNK = EPT // EC
GPC = EC * 4 // 16
NSLICE = 8
SLICE_N = NPAD // NSLICE
PORT_N = SLICE_N // NTILES
PORT_W = PORT_N * 4
SENT = 1e30
MASK_HI = -65536


def _build():
    mesh = plsc.VectorSubcoreMesh(core_axis_name="c", subcore_axis_name="s")

    @functools.partial(
        pl.kernel,
        mesh=mesh,
        compiler_params=pltpu.CompilerParams(needs_layout_passes=False),
        out_type=[
            jax.ShapeDtypeStruct((N_NODES, N_UNARY), jnp.float32),
            jax.ShapeDtypeStruct((N_EDGES * 4,), jnp.float32),
        ],
        scratch_types=[
            pltpu.VMEM((NPAD * 2,), jnp.int32),
            pltpu.VMEM((TFLAT,), jnp.float32),
            pltpu.VMEM((TFLAT,), jnp.float32),
            pltpu.VMEM((P1C, N_UNARY), jnp.float32),
            pltpu.VMEM((NSLICE * PORT_W,), jnp.float32),
            pltpu.VMEM((SPAN * 2,), jnp.int32),
            pltpu.VMEM((NTILES * PORT_W,), jnp.float32),
            pltpu.VMEM((EC,), jnp.int32),
            pltpu.VMEM((EC,), jnp.int32),
            pltpu.VMEM((EC,), jnp.float32),
            pltpu.VMEM((EC * 4,), jnp.float32),
            pltpu.VMEM((EC * 4,), jnp.float32),
            pltpu.VMEM((16,), jnp.float32),
            pltpu.VMEM((16,), jnp.float32),
            pltpu.VMEM_SHARED((NTILES * SPAN * 2,), jnp.int32),
            pltpu.VMEM_SHARED((NTILES * 2 * NTILES * PORT_W,), jnp.float32),
            pltpu.SemaphoreType.DMA,
        ],
    )
    def k(unary, idx1, idx2, ew, binary, wlane, wb16, out_u, out_b,
          u4p, t1, t2, rowc, piece, u4piece, comb, i1c, i2c, ewc, binc, boc,
          wl_v, wb_v, u4_sh, sb, sem):
        core = lax.axis_index("c")
        w = lax.axis_index("s")

        @pl.when(core == 0)
        def _body():
            iota = lax.iota(jnp.int32, 16)
            pltpu.sync_copy(wlane, wl_v)
            pltpu.sync_copy(wb16, wb_v)
            wlv = wl_v[...]
            slane = (2 * (iota & 1) - 1).astype(jnp.float32)

            def enhance_row(r):
                rr = jnp.full((16,), r, jnp.int32)
                v = plsc.load_gather(rowc, [rr, iota])
                vsw = plsc.load_gather(rowc, [rr, iota ^ 1])
                arg = slane * (v + vsw)
                sig = 1.0 / (1.0 + jnp.exp(-arg))
                return v, vsw, sig

            nch1 = jnp.where(w == 15, SPAN_LAST // P1C, SPAN // P1C)
            row0 = w * SPAN

            def p1_chunk(kk, _):
                base = kk * P1C
                pltpu.sync_copy(unary.at[pl.ds(row0 + base, P1C), :], rowc)

                def p1_row(r, _):
                    v, vsw, sig = enhance_row(r)
                    u16 = v + wlv * sig
                    u16s = vsw - wlv * (1.0 - sig)
                    lo = lax.shift_right_logical(plsc.bitcast(u16, jnp.int32), 16)
                    hi = plsc.bitcast(u16s, jnp.int32) & MASK_HI
                    plsc.store_scatter(u4piece, [(base + r) * 2 + (iota >> 1)],
                                       lo | hi, mask=(iota & 1) == 0)
                    return _

                lax.fori_loop(0, P1C, p1_row, 0)
                return _

            lax.fori_loop(0, nch1, p1_chunk, 0)
            pltpu.sync_copy(u4piece, u4_sh.at[pl.ds(w * SPAN * 2, SPAN * 2)])
            plsc.subcore_barrier()
            pltpu.sync_copy(u4_sh, u4p)

            def tinit(g, _):
                t1[pl.ds(g * 16, 16)] = jnp.full((16,), SENT, jnp.float32)
                t2[pl.ds(g * 16, 16)] = jnp.full((16,), SENT, jnp.float32)
                return _

            lax.fori_loop(0, TFLAT // 16, tinit, 0)

            wbv = wb_v[...]
            e_of = iota >> 2
            c_of = iota & 3
            lowhalf = (c_of & 1) == 0
            pair = c_of >> 1

            def unpack(ref, i, p):
                word = plsc.load_gather(ref, [i * 2 + p])
                bits = jnp.where(lowhalf, lax.shift_left(word, 16), word & MASK_HI)
                return plsc.bitcast(bits, jnp.float32)

            def e_chunk(kk, _):
                base = w * EPT + kk * EC
                cp1 = pltpu.async_copy(idx1.at[pl.ds(base, EC)], i1c, sem)
                cp2 = pltpu.async_copy(idx2.at[pl.ds(base, EC)], i2c, sem)
                cp3 = pltpu.async_copy(ew.at[pl.ds(base, EC)], ewc, sem)
                cp4 = pltpu.async_copy(binary.at[pl.ds(base * 4, EC * 4)], binc, sem)
                cp1.wait(); cp2.wait(); cp3.wait(); cp4.wait()

                def e_group(g, _):
                    e = g * 4 + e_of
                    i1 = plsc.load_gather(i1c, [e])
                    i2 = plsc.load_gather(i2c, [e])
                    wv = plsc.load_gather(ewc, [e])
                    x4 = unpack(u4p, i1, pair)
                    y4 = unpack(u4p, i2, pair)
                    bv = binc[pl.ds(g * 16, 16)]
                    m = jnp.maximum(jnp.maximum(-x4, bv), y4)
                    ea = jnp.exp(-x4 - m)
                    eb = jnp.exp(bv - m)
                    ec = jnp.exp(y4 - m)
                    r = wbv * wv / (ea + eb + ec)
                    boc[pl.ds(g * 16, 16)] = bv + r * eb
                    plsc.store_scatter(t1, [i1 * 4 + c_of], -r * ea)
                    plsc.store_scatter(t2, [i2 * 4 + c_of], r * ec)
                    return _

                lax.fori_loop(0, GPC, e_group, 0)
                pltpu.sync_copy(boc, out_b.at[pl.ds(base * 4, EC * 4)])
                return _

            lax.fori_loop(0, NCHUNK, e_chunk, 0)

            def c_slice(s, _):
                src0 = s * (SLICE_N * 4)
                for o in range(NTILES):
                    pltpu.sync_copy(
                        t1.at[pl.ds(src0 + o * PORT_W, PORT_W)],
                        sb.at[pl.ds(((o * 2 + 0) * NTILES + w) * PORT_W, PORT_W)])
                    pltpu.sync_copy(
                        t2.at[pl.ds(src0 + o * PORT_W, PORT_W)],
                        sb.at[pl.ds(((o * 2 + 1) * NTILES + w) * PORT_W, PORT_W)])
                plsc.subcore_barrier()

                def c_side(side, accum):
                    pltpu.sync_copy(sb.at[pl.ds((w * 2 + side) * NTILES * PORT_W,
                                                NTILES * PORT_W)], comb)

                    def c_group(g, _):
                        acc = jnp.full((16,), SENT, jnp.float32)
                        for t in range(NTILES):
                            v = comb[pl.ds(t * PORT_W + g * 16, 16)]
                            acc = jnp.where(v == SENT, acc, v)
                        d = jnp.where(acc == SENT, 0.0, acc)
                        dst = pl.ds(s * PORT_W + g * 16, 16)
                        piece[dst] = d if not accum else piece[dst] + d
                        return _

                    lax.fori_loop(0, PORT_W // 16, c_group, 0)

                c_side(0, False)
                c_side(1, True)
                plsc.subcore_barrier()
                return _

            lax.fori_loop(0, NSLICE, c_slice, 0)

            def p3_portion(s, _):
                prow = s * SLICE_N + w * PORT_N

                @pl.when(prow < N_NODES)
                def _emit():
                    def p3_chunk(kk, _):
                        base = kk * P1C
                        pltpu.sync_copy(unary.at[pl.ds(prow + base, P1C), :], rowc)

                        def p3_row(r, _):
                            v, vsw, sig = enhance_row(r)
                            u16 = v + wlv * sig
                            f4 = plsc.load_gather(
                                piece, [s * PORT_W + (base + r) * 4 + c_of])
                            plsc.store_scatter(
                                rowc, [jnp.full((16,), r, jnp.int32), iota],
                                jnp.where(iota < 4, u16 + f4, u16))
                            return _

                        lax.fori_loop(0, P1C, p3_row, 0)
                        pltpu.sync_copy(rowc, out_u.at[pl.ds(prow + base, P1C), :])
                        return _

                    lax.fori_loop(0, PORT_N // P1C, p3_chunk, 0)

                return _

            lax.fori_loop(0, NSLICE, p3_portion, 0)

    return k


def kernel(unary, binary, edge_index, edge_weight, unary_clause_weights, binary_clause_weights):
    idx1 = edge_index[0].astype(jnp.int32)
    idx2 = edge_index[1].astype(jnp.int32)
    wl = jnp.stack([-unary_clause_weights, unary_clause_weights], axis=1).reshape(16)
    wb16 = jnp.tile(binary_clause_weights, 4)
    out_u, out_b = _build()(unary, idx1, idx2, edge_weight,
                            binary.reshape(-1), wl, wb16)
    return (out_u, out_b.reshape(N_EDGES, 4))


# speedup vs baseline: 2.6838x; 2.6838x over previous
import functools

import jax
import jax.numpy as jnp
from jax import lax
from jax.experimental import pallas as pl
from jax.experimental.pallas import tpu as pltpu
from jax.experimental.pallas import tpu_sc as plsc

N_NODES = 10000
N_EDGES = 160000
N_UNARY = 128

NTILES = 16
SPAN = 640
SPAN_LAST = N_NODES - 15 * SPAN
NPAD = NTILES * SPAN
TFLAT = NPAD * 4
EHALF = N_EDGES // 2
EPT = EHALF // NTILES
EC = 200
NCHUNK = EPT // EC
GPC = EC * 4 // 16
NSLICE = 16
SLICE_N = NPAD // NSLICE
PORT_N = SLICE_N // NTILES
PORT_W = PORT_N * 4
DHALF = NSLICE * PORT_W
SENT = 1e30
MASK_HI = -65536


def _build_sc():
    mesh = plsc.VectorSubcoreMesh(core_axis_name="c", subcore_axis_name="s")

    @functools.partial(
        pl.kernel,
        mesh=mesh,
        compiler_params=pltpu.CompilerParams(needs_layout_passes=False),
        out_type=[
            jax.ShapeDtypeStruct((NPAD * 8,), jnp.float32),
            jax.ShapeDtypeStruct((NPAD * 8,), jnp.float32),
            jax.ShapeDtypeStruct((N_EDGES * 4,), jnp.float32),
        ],
        scratch_types=[
            pltpu.VMEM((NPAD * 2,), jnp.int32),
            pltpu.VMEM((TFLAT,), jnp.float32),
            pltpu.VMEM((TFLAT,), jnp.float32),
            pltpu.VMEM((5120,), jnp.float32),
            pltpu.VMEM((2 * DHALF,), jnp.float32),
            pltpu.VMEM((SPAN * 2,), jnp.int32),
            pltpu.VMEM((NTILES * PORT_W,), jnp.float32),
            pltpu.VMEM((EC,), jnp.int32),
            pltpu.VMEM((EC,), jnp.int32),
            pltpu.VMEM((EC,), jnp.float32),
            pltpu.VMEM((EC * 4,), jnp.float32),
            pltpu.VMEM((EC * 4,), jnp.float32),
            pltpu.VMEM((EC,), jnp.int32),
            pltpu.VMEM((EC,), jnp.int32),
            pltpu.VMEM((EC,), jnp.float32),
            pltpu.VMEM((EC * 4,), jnp.float32),
            pltpu.VMEM((EC * 4,), jnp.float32),
            pltpu.VMEM((16,), jnp.float32),
            pltpu.VMEM((16,), jnp.float32),
            pltpu.VMEM_SHARED((NTILES * SPAN * 2,), jnp.int32),
            pltpu.VMEM_SHARED((NTILES * 2 * NTILES * PORT_W,), jnp.float32),
            pltpu.SemaphoreType.DMA,
            pltpu.SemaphoreType.DMA,
            pltpu.SemaphoreType.DMA,
        ],
    )
    def k(u16f, idx1, idx2, ew, binary, wlane, wb16, out_d0, out_d1, out_b,
          u4p, t1, t2, rowc, piece, u4piece, comb,
          i1cA, i2cA, ewcA, bincA, bocA, i1cB, i2cB, ewcB, bincB, bocB,
          wl_v, wb_v, u4_sh, sb, sem, semA, semB):
        core = lax.axis_index("c")
        w = lax.axis_index("s")
        iota = lax.iota(jnp.int32, 16)
        pltpu.sync_copy(wlane, wl_v)
        pltpu.sync_copy(wb16, wb_v)
        wlv = wl_v[...]
        slane = (2 * (iota & 1) - 1).astype(jnp.float32)

        rows_w = jnp.where(w == 15, SPAN_LAST, SPAN)
        row0 = w * SPAN

        def p1_chunk(kk, _):
            base = jnp.minimum(kk * 320, rows_w - 320)
            pltpu.sync_copy(u16f.at[pl.ds((row0 + base) * 16, 5120)], rowc)

            def p1_row(r, _):
                v = rowc[pl.ds(r * 16, 16)]
                vsw = plsc.load_gather(rowc, [r * 16 + (iota ^ 1)])
                arg = slane * (v + vsw)
                sig = 1.0 / (1.0 + jnp.exp(-arg))
                u16 = v + wlv * sig
                u16s = vsw - wlv * (1.0 - sig)
                lo = lax.shift_right_logical(plsc.bitcast(u16, jnp.int32), 16)
                hi = plsc.bitcast(u16s, jnp.int32) & MASK_HI
                plsc.store_scatter(u4piece, [(base + r) * 2 + (iota >> 1)],
                                   lo | hi, mask=(iota & 1) == 0)
                return _

            lax.fori_loop(0, 320, p1_row, 0)
            return _

        lax.fori_loop(0, 2, p1_chunk, 0)
        pltpu.sync_copy(u4piece, u4_sh.at[pl.ds(w * SPAN * 2, SPAN * 2)])

        @pl.when(w == 0)
        def _sent_fill():
            def sfill(g, _):
                piece[pl.ds(g * 16, 16)] = jnp.full((16,), SENT, jnp.float32)
                return _

            lax.fori_loop(0, DHALF // 16, sfill, 0)
            pltpu.sync_copy(piece.at[pl.ds(0, DHALF)], sb.at[pl.ds(0, DHALF)])

        plsc.subcore_barrier()
        pltpu.sync_copy(u4_sh, u4p)

        sent_src = sb.at[pl.ds(0, DHALF)]
        icopies = []
        for i in range(TFLAT // DHALF):
            dst = pl.ds(i * DHALF, DHALF)
            icopies.append(pltpu.async_copy(sent_src, t1.at[dst], sem))
            icopies.append(pltpu.async_copy(sent_src, t2.at[dst], sem))
        for c in icopies:
            c.wait()

        wbv = wb_v[...]
        e_of = iota >> 2
        c_of = iota & 3
        lowhalf = (c_of & 1) == 0
        pair = c_of >> 1
        ebase0 = core * EHALF + w * EPT

        def unpack(ref, i, p):
            word = plsc.load_gather(ref, [i * 2 + p])
            bits = jnp.where(lowhalf, lax.shift_left(word, 16), word & MASK_HI)
            return plsc.bitcast(bits, jnp.float32)

        setA = (i1cA, i2cA, ewcA, bincA, bocA, semA)
        setB = (i1cB, i2cB, ewcB, bincB, bocB, semB)

        def e_copies(st, kk):
            i1c, i2c, ewc, binc, boc, sm = st
            base = ebase0 + kk * EC
            return (
                pltpu.make_async_copy(idx1.at[pl.ds(base, EC)], i1c, sm),
                pltpu.make_async_copy(idx2.at[pl.ds(base, EC)], i2c, sm),
                pltpu.make_async_copy(ew.at[pl.ds(base, EC)], ewc, sm),
            ) + tuple(
                pltpu.make_async_copy(binary.at[pl.ds(c * N_EDGES + base, EC)],
                                      binc.at[pl.ds(c * EC, EC)], sm)
                for c in range(4)
            )

        def e_fire(st, kk):
            for c in e_copies(st, kk):
                c.start()

        def e_drain(st, kk):
            for c in e_copies(st, kk):
                c.wait()

        def e_compute(st, kk):
            i1c, i2c, ewc, binc, boc, sm = st
            base = ebase0 + kk * EC

            def e_group(g, _):
                e = g * 4 + e_of
                i1 = plsc.load_gather(i1c, [e])
                i2 = plsc.load_gather(i2c, [e])
                wv = plsc.load_gather(ewc, [e])
                x4 = unpack(u4p, i1, pair)
                y4 = unpack(u4p, i2, pair)
                bv = plsc.load_gather(binc, [c_of * EC + e])
                m = jnp.maximum(jnp.maximum(-x4, bv), y4)
                ea = jnp.exp(-x4 - m)
                eb = jnp.exp(bv - m)
                ec = jnp.exp(y4 - m)
                r = wbv * wv / (ea + eb + ec)
                plsc.store_scatter(boc, [c_of * EC + e], bv + r * eb)
                plsc.store_scatter(t1, [i1 * 4 + c_of], -r * ea)
                plsc.store_scatter(t2, [i2 * 4 + c_of], r * ec)
                return _

            lax.fori_loop(0, GPC, e_group, 0)
            owrites = [pltpu.async_copy(boc.at[pl.ds(c * EC, EC)],
                                        out_b.at[pl.ds(c * N_EDGES + base, EC)], sm)
                       for c in range(4)]
            for c in owrites:
                c.wait()

        e_fire(setA, 0)

        def e_pair(j, _):
            e_fire(setB, 2 * j + 1)
            e_drain(setA, 2 * j)
            e_compute(setA, 2 * j)
            e_fire(setA, 2 * j + 2)
            e_drain(setB, 2 * j + 1)
            e_compute(setB, 2 * j + 1)
            return _

        lax.fori_loop(0, (NCHUNK - 1) // 2, e_pair, 0)
        e_drain(setA, NCHUNK - 1)
        e_compute(setA, NCHUNK - 1)

        def c_slice(s, _):
            src0 = s * (SLICE_N * 4)
            stage = []
            for o in range(NTILES):
                stage.append(pltpu.async_copy(
                    t1.at[pl.ds(src0 + o * PORT_W, PORT_W)],
                    sb.at[pl.ds(((o * 2 + 0) * NTILES + w) * PORT_W, PORT_W)], sem))
                stage.append(pltpu.async_copy(
                    t2.at[pl.ds(src0 + o * PORT_W, PORT_W)],
                    sb.at[pl.ds(((o * 2 + 1) * NTILES + w) * PORT_W, PORT_W)], sem))
            for c in stage:
                c.wait()
            plsc.subcore_barrier()

            def c_side(side):
                pltpu.sync_copy(sb.at[pl.ds((w * 2 + side) * NTILES * PORT_W,
                                            NTILES * PORT_W)], comb)

                def c_group(g, _):
                    acc = jnp.full((16,), SENT, jnp.float32)
                    for t in range(NTILES):
                        v = comb[pl.ds(t * PORT_W + g * 16, 16)]
                        acc = jnp.where(v == SENT, acc, v)
                    piece[pl.ds(side * DHALF + s * PORT_W + g * 16, 16)] = acc
                    return _

                lax.fori_loop(0, PORT_W // 16, c_group, 0)

            c_side(0)
            c_side(1)
            plsc.subcore_barrier()
            return _

        lax.fori_loop(0, NSLICE, c_slice, 0)

        def emit(out_d):
            dcopies = []
            for side in range(2):
                for s in range(NSLICE):
                    dcopies.append(pltpu.async_copy(
                        piece.at[pl.ds(side * DHALF + s * PORT_W, PORT_W)],
                        out_d.at[pl.ds(side * TFLAT + s * SLICE_N * 4 + w * PORT_W,
                                       PORT_W)], sem))
            for c in dcopies:
                c.wait()

        @pl.when(core == 0)
        def _emit0():
            emit(out_d0)

        @pl.when(core == 1)
        def _emit1():
            emit(out_d1)

    return k


def _tc_assemble():
    def body(u_ref, a1_ref, a2_ref, b1_ref, b2_ref, wl_ref, o_ref):
        x = u_ref[...]
        v16 = x[:, :16]
        i2 = lax.broadcasted_iota(jnp.int32, (16, 16), 0)
        j2 = lax.broadcasted_iota(jnp.int32, (16, 16), 1)
        perm = (j2 == (i2 ^ 1)).astype(jnp.float32)
        vsw = jnp.dot(v16, perm, preferred_element_type=jnp.float32)
        lane = lax.broadcasted_iota(jnp.int32, (1, 16), 1)
        slane = (2 * (lane & 1) - 1).astype(jnp.float32)
        arg = slane * (v16 + vsw)
        sig = 1.0 / (1.0 + jnp.exp(-arg))
        u16 = v16 + wl_ref[...] * sig

        def pick(a, b):
            d = jnp.where(b == SENT, a, b)
            return jnp.where(d == SENT, 0.0, d)

        d = pick(a1_ref[...], b1_ref[...]) + pick(a2_ref[...], b2_ref[...])
        o16 = jnp.concatenate([u16[:, :4] + d, u16[:, 4:]], axis=1)
        o_ref[...] = jnp.concatenate([o16, x[:, 16:]], axis=1)

    blk = 400
    dspec = pl.BlockSpec((blk, 4), lambda i: (i, 0))
    return pl.pallas_call(
        body,
        grid=(N_NODES // blk,),
        in_specs=[
            pl.BlockSpec((blk, N_UNARY), lambda i: (i, 0)),
            dspec, dspec, dspec, dspec,
            pl.BlockSpec((1, 16), lambda i: (0, 0)),
        ],
        out_specs=pl.BlockSpec((blk, N_UNARY), lambda i: (i, 0)),
        out_shape=jax.ShapeDtypeStruct((N_NODES, N_UNARY), jnp.float32),
    )


def kernel(unary, binary, edge_index, edge_weight, unary_clause_weights, binary_clause_weights):
    idx1 = edge_index[0].astype(jnp.int32)
    idx2 = edge_index[1].astype(jnp.int32)
    wl = jnp.stack([-unary_clause_weights, unary_clause_weights], axis=1).reshape(16)
    wb16 = jnp.tile(binary_clause_weights, 4)
    u16f = unary[:, :16].reshape(-1)
    out_d0, out_d1, out_b = _build_sc()(u16f, idx1, idx2, edge_weight,
                                        binary.T.reshape(-1), wl, wb16)
    da = out_d0.reshape(2, NPAD, 4)
    db = out_d1.reshape(2, NPAD, 4)
    out_u = _tc_assemble()(unary, da[0, :N_NODES], da[1, :N_NODES],
                           db[0, :N_NODES], db[1, :N_NODES], wl.reshape(1, 16))
    return (out_u, out_b.reshape(4, N_EDGES).T)


# speedup vs baseline: 3.2968x; 1.2284x over previous
import functools

import jax
import jax.numpy as jnp
from jax import lax
from jax.experimental import pallas as pl
from jax.experimental.pallas import tpu as pltpu
from jax.experimental.pallas import tpu_sc as plsc

N_NODES = 10000
N_EDGES = 160000
N_UNARY = 128

NTILES = 16
SPAN = 640
SPAN_LAST = N_NODES - 15 * SPAN
NPAD = NTILES * SPAN
TFLAT = NPAD * 4
EHALF = N_EDGES // 2
EPT = EHALF // NTILES
EC = 200
NCHUNK = EPT // EC
GPC = EC * 4 // 16
NSLICE = 16
SLICE_N = NPAD // NSLICE
PORT_N = SLICE_N // NTILES
PORT_W = PORT_N * 4
DHALF = NSLICE * PORT_W
SENT = 1e30
MASK_HI = -65536


def _build_sc():
    mesh = plsc.VectorSubcoreMesh(core_axis_name="c", subcore_axis_name="s")

    @functools.partial(
        pl.kernel,
        mesh=mesh,
        compiler_params=pltpu.CompilerParams(needs_layout_passes=False),
        out_type=[
            jax.ShapeDtypeStruct((NPAD * 8,), jnp.float32),
            jax.ShapeDtypeStruct((NPAD * 8,), jnp.float32),
            jax.ShapeDtypeStruct((N_EDGES * 4,), jnp.float32),
        ],
        scratch_types=[
            pltpu.VMEM((NPAD * 2,), jnp.int32),
            pltpu.VMEM((TFLAT,), jnp.float32),
            pltpu.VMEM((TFLAT,), jnp.float32),
            pltpu.VMEM((5120,), jnp.float32),
            pltpu.VMEM((2 * DHALF,), jnp.float32),
            pltpu.VMEM((SPAN * 2,), jnp.int32),
            pltpu.VMEM((NTILES * PORT_W,), jnp.float32),
            pltpu.VMEM((EC,), jnp.int32),
            pltpu.VMEM((EC,), jnp.int32),
            pltpu.VMEM((EC,), jnp.float32),
            pltpu.VMEM((EC * 4,), jnp.float32),
            pltpu.VMEM((EC * 4,), jnp.float32),
            pltpu.VMEM((EC,), jnp.int32),
            pltpu.VMEM((EC,), jnp.int32),
            pltpu.VMEM((EC,), jnp.float32),
            pltpu.VMEM((EC * 4,), jnp.float32),
            pltpu.VMEM((EC * 4,), jnp.float32),
            pltpu.VMEM((16,), jnp.float32),
            pltpu.VMEM((16,), jnp.float32),
            pltpu.VMEM_SHARED((NTILES * SPAN * 2,), jnp.int32),
            pltpu.VMEM_SHARED((NTILES * 2 * NTILES * PORT_W,), jnp.float32),
            pltpu.SemaphoreType.DMA,
            pltpu.SemaphoreType.DMA,
            pltpu.SemaphoreType.DMA,
        ],
    )
    def k(u16f, idx1, idx2, ew, binary, wlane, wb16, out_d0, out_d1, out_b,
          u4p, t1, t2, rowc, piece, u4piece, comb,
          i1cA, i2cA, ewcA, bincA, bocA, i1cB, i2cB, ewcB, bincB, bocB,
          wl_v, wb_v, u4_sh, sb, sem, semA, semB):
        core = lax.axis_index("c")
        w = lax.axis_index("s")
        iota = lax.iota(jnp.int32, 16)
        pltpu.sync_copy(wlane, wl_v)
        pltpu.sync_copy(wb16, wb_v)
        wlv = wl_v[...]
        slane = (2 * (iota & 1) - 1).astype(jnp.float32)

        rows_w = jnp.where(w == 15, SPAN_LAST, SPAN)
        row0 = w * SPAN

        def p1_chunk(kk, _):
            base = jnp.minimum(kk * 320, rows_w - 320)
            pltpu.sync_copy(u16f.at[pl.ds((row0 + base) * 16, 5120)], rowc)

            def p1_row(r, _):
                v = rowc[pl.ds(r * 16, 16)]
                vsw = plsc.load_gather(rowc, [r * 16 + (iota ^ 1)])
                arg = slane * (v + vsw)
                sig = 1.0 / (1.0 + jnp.exp(-arg))
                u16 = v + wlv * sig
                u16s = vsw - wlv * (1.0 - sig)
                lo = lax.shift_right_logical(plsc.bitcast(u16, jnp.int32), 16)
                hi = plsc.bitcast(u16s, jnp.int32) & MASK_HI
                plsc.store_scatter(u4piece, [(base + r) * 2 + (iota >> 1)],
                                   lo | hi, mask=(iota & 1) == 0)
                return _

            lax.fori_loop(0, 320, p1_row, 0)
            return _

        lax.fori_loop(0, 2, p1_chunk, 0)
        pltpu.sync_copy(u4piece, u4_sh.at[pl.ds(w * SPAN * 2, SPAN * 2)])

        @pl.when(w == 0)
        def _sent_fill():
            def sfill(g, _):
                piece[pl.ds(g * 16, 16)] = jnp.full((16,), SENT, jnp.float32)
                return _

            lax.fori_loop(0, DHALF // 16, sfill, 0)
            pltpu.sync_copy(piece.at[pl.ds(0, DHALF)], sb.at[pl.ds(0, DHALF)])

        plsc.subcore_barrier()
        pltpu.sync_copy(u4_sh, u4p)

        sent_src = sb.at[pl.ds(0, DHALF)]
        icopies = []
        for i in range(TFLAT // DHALF):
            dst = pl.ds(i * DHALF, DHALF)
            icopies.append(pltpu.async_copy(sent_src, t1.at[dst], sem))
            icopies.append(pltpu.async_copy(sent_src, t2.at[dst], sem))
        for c in icopies:
            c.wait()

        wbv = wb_v[...]
        e_of = iota >> 2
        c_of = iota & 3
        lowhalf = (c_of & 1) == 0
        pair = c_of >> 1
        ebase0 = core * EHALF + w * EPT

        def unpack(ref, i, p):
            word = plsc.load_gather(ref, [i * 2 + p])
            bits = jnp.where(lowhalf, lax.shift_left(word, 16), word & MASK_HI)
            return plsc.bitcast(bits, jnp.float32)

        setA = (i1cA, i2cA, ewcA, bincA, bocA, semA)
        setB = (i1cB, i2cB, ewcB, bincB, bocB, semB)

        def e_copies(st, kk):
            i1c, i2c, ewc, binc, boc, sm = st
            base = ebase0 + kk * EC
            return (
                pltpu.make_async_copy(idx1.at[pl.ds(base, EC)], i1c, sm),
                pltpu.make_async_copy(idx2.at[pl.ds(base, EC)], i2c, sm),
                pltpu.make_async_copy(ew.at[pl.ds(base, EC)], ewc, sm),
            ) + tuple(
                pltpu.make_async_copy(binary.at[pl.ds(c * N_EDGES + base, EC)],
                                      binc.at[pl.ds(c * EC, EC)], sm)
                for c in range(4)
            )

        def e_fire(st, kk):
            for c in e_copies(st, kk):
                c.start()

        def e_drain(st, kk):
            for c in e_copies(st, kk):
                c.wait()

        def e_compute(st, kk):
            i1c, i2c, ewc, binc, boc, sm = st
            base = ebase0 + kk * EC

            def e_group(g, _):
                e = g * 4 + e_of
                i1 = plsc.load_gather(i1c, [e])
                i2 = plsc.load_gather(i2c, [e])
                wv = plsc.load_gather(ewc, [e])
                x4 = unpack(u4p, i1, pair)
                y4 = unpack(u4p, i2, pair)
                bv = plsc.load_gather(binc, [c_of * EC + e])
                m = jnp.maximum(jnp.maximum(-x4, bv), y4)
                ea = jnp.exp(-x4 - m)
                eb = jnp.exp(bv - m)
                ec = jnp.exp(y4 - m)
                r = wbv * wv / (ea + eb + ec)
                plsc.store_scatter(boc, [c_of * EC + e], bv + r * eb)
                plsc.store_scatter(t1, [i1 * 4 + c_of], -r * ea)
                plsc.store_scatter(t2, [i2 * 4 + c_of], r * ec)
                return _

            lax.fori_loop(0, GPC, e_group, 0)
            owrites = [pltpu.async_copy(boc.at[pl.ds(c * EC, EC)],
                                        out_b.at[pl.ds(c * N_EDGES + base, EC)], sm)
                       for c in range(4)]
            for c in owrites:
                c.wait()

        e_fire(setA, 0)

        def e_pair(j, _):
            e_fire(setB, 2 * j + 1)
            e_drain(setA, 2 * j)
            e_compute(setA, 2 * j)
            e_fire(setA, 2 * j + 2)
            e_drain(setB, 2 * j + 1)
            e_compute(setB, 2 * j + 1)
            return _

        lax.fori_loop(0, (NCHUNK - 1) // 2, e_pair, 0)
        e_drain(setA, NCHUNK - 1)
        e_compute(setA, NCHUNK - 1)

        def c_slice(s, _):
            src0 = s * (SLICE_N * 4)
            stage = []
            for o in range(NTILES):
                stage.append(pltpu.async_copy(
                    t1.at[pl.ds(src0 + o * PORT_W, PORT_W)],
                    sb.at[pl.ds(((o * 2 + 0) * NTILES + w) * PORT_W, PORT_W)], sem))
                stage.append(pltpu.async_copy(
                    t2.at[pl.ds(src0 + o * PORT_W, PORT_W)],
                    sb.at[pl.ds(((o * 2 + 1) * NTILES + w) * PORT_W, PORT_W)], sem))
            for c in stage:
                c.wait()
            plsc.subcore_barrier()

            def c_side(side):
                pltpu.sync_copy(sb.at[pl.ds((w * 2 + side) * NTILES * PORT_W,
                                            NTILES * PORT_W)], comb)

                def c_group(g, _):
                    acc = jnp.full((16,), SENT, jnp.float32)
                    for t in range(NTILES):
                        v = comb[pl.ds(t * PORT_W + g * 16, 16)]
                        acc = jnp.where(v == SENT, acc, v)
                    f = g * 16 + iota
                    plsc.store_scatter(
                        piece, [(s * PORT_N + (f >> 2)) * 8 + side * 4 + (f & 3)], acc)
                    return _

                lax.fori_loop(0, PORT_W // 16, c_group, 0)

            c_side(0)
            c_side(1)
            plsc.subcore_barrier()
            return _

        lax.fori_loop(0, NSLICE, c_slice, 0)

        def emit(out_d):
            dcopies = []
            for s in range(NSLICE):
                dcopies.append(pltpu.async_copy(
                    piece.at[pl.ds(s * PORT_N * 8, PORT_N * 8)],
                    out_d.at[pl.ds((s * SLICE_N + w * PORT_N) * 8, PORT_N * 8)], sem))
            for c in dcopies:
                c.wait()

        @pl.when(core == 0)
        def _emit0():
            emit(out_d0)

        @pl.when(core == 1)
        def _emit1():
            emit(out_d1)

    return k


def _tc_assemble():
    def body(u_ref, a_ref, b_ref, wl_ref, o_ref):
        x = u_ref[...]
        v16 = x[:, :16]
        i2 = lax.broadcasted_iota(jnp.int32, (16, 16), 0)
        j2 = lax.broadcasted_iota(jnp.int32, (16, 16), 1)
        perm = (j2 == (i2 ^ 1)).astype(jnp.float32)
        vsw = jnp.dot(v16, perm, preferred_element_type=jnp.float32)
        lane = lax.broadcasted_iota(jnp.int32, (1, 16), 1)
        slane = (2 * (lane & 1) - 1).astype(jnp.float32)
        arg = slane * (v16 + vsw)
        sig = 1.0 / (1.0 + jnp.exp(-arg))
        u16 = v16 + wl_ref[...] * sig

        def pick(a, b):
            d = jnp.where(b == SENT, a, b)
            return jnp.where(d == SENT, 0.0, d)

        a = a_ref[...]
        b = b_ref[...]
        d = pick(a[:, 0:4], b[:, 0:4]) + pick(a[:, 4:8], b[:, 4:8])
        o16 = jnp.concatenate([u16[:, :4] + d, u16[:, 4:]], axis=1)
        o_ref[...] = jnp.concatenate([o16, x[:, 16:]], axis=1)

    blk = 1000
    dspec = pl.BlockSpec((blk, 8), lambda i: (i, 0))
    return pl.pallas_call(
        body,
        grid=(N_NODES // blk,),
        in_specs=[
            pl.BlockSpec((blk, N_UNARY), lambda i: (i, 0)),
            dspec, dspec,
            pl.BlockSpec((1, 16), lambda i: (0, 0)),
        ],
        out_specs=pl.BlockSpec((blk, N_UNARY), lambda i: (i, 0)),
        out_shape=jax.ShapeDtypeStruct((N_NODES, N_UNARY), jnp.float32),
    )


def kernel(unary, binary, edge_index, edge_weight, unary_clause_weights, binary_clause_weights):
    idx1 = edge_index[0].astype(jnp.int32)
    idx2 = edge_index[1].astype(jnp.int32)
    wl = jnp.stack([-unary_clause_weights, unary_clause_weights], axis=1).reshape(16)
    wb16 = jnp.tile(binary_clause_weights, 4)
    u16f = unary[:, :16].reshape(-1)
    out_d0, out_d1, out_b = _build_sc()(u16f, idx1, idx2, edge_weight,
                                        binary.T.reshape(-1), wl, wb16)
    da = out_d0.reshape(NPAD, 8)[:N_NODES]
    db = out_d1.reshape(NPAD, 8)[:N_NODES]
    out_u = _tc_assemble()(unary, da, db, wl.reshape(1, 16))
    return (out_u, out_b.reshape(4, N_EDGES).T)
